# Initial kernel scaffold; baseline (speedup 1.0000x reference)
import functools

import jax
import jax.numpy as jnp
from jax import lax
from jax.experimental import pallas as pl
from jax.experimental.pallas import tpu as pltpu
from jax.experimental.pallas import tpu_sc as plsc

F32 = jnp.float32

NC = 2
NS = 16
NW = NC * NS
CH = 128


def _round_up(a, m):
    return (a + m - 1) // m * m


def _tables_body(x_ref, pos_ref, wa_ref, wb_ref, b1_ref, gs_ref, gr_ref):
    xb = x_ref[...]
    bn = xb.shape[0]
    pz = jnp.concatenate(
        [pos_ref[...], jnp.zeros((bn, 13), F32)], axis=1)
    ms = jnp.dot(xb, wa_ref[...], preferred_element_type=F32)
    mr = jnp.dot(xb, wb_ref[...], preferred_element_type=F32) + b1_ref[...]
    gs_ref[...] = jnp.concatenate([ms, pz], axis=1)
    gr_ref[...] = jnp.concatenate([mr, -pz], axis=1)


def _build_tables(x, pos, waT, wbT, b1):
    n, d = x.shape
    bn = 1000
    grid = n // bn
    return pl.pallas_call(
        _tables_body,
        grid=(grid,),
        in_specs=[
            pl.BlockSpec((bn, d), lambda i: (i, 0)),
            pl.BlockSpec((bn, 3), lambda i: (i, 0)),
            pl.BlockSpec((d, d), lambda i: (0, 0)),
            pl.BlockSpec((d, d), lambda i: (0, 0)),
            pl.BlockSpec((1, d), lambda i: (0, 0)),
        ],
        out_specs=[
            pl.BlockSpec((bn, d + 16), lambda i: (i, 0)),
            pl.BlockSpec((bn, d + 16), lambda i: (i, 0)),
        ],
        out_shape=[
            jax.ShapeDtypeStruct((n, d + 16), F32),
            jax.ShapeDtypeStruct((n, d + 16), F32),
        ],
    )(x, pos, waT, wbT, b1)


def _gather_kernel(e_pad, n, dw):
    epw = e_pad // NW
    nchunk = epw // CH
    mesh = plsc.VectorSubcoreMesh(
        core_axis_name="c", subcore_axis_name="s",
        num_cores=NC, num_subcores=NS)

    @functools.partial(
        pl.kernel,
        out_type=jax.ShapeDtypeStruct((e_pad, dw), F32),
        mesh=mesh,
        scratch_types=[
            pltpu.VMEM((CH,), jnp.int32),
            pltpu.VMEM((CH,), jnp.int32),
            pltpu.VMEM((CH, dw), F32),
            pltpu.VMEM((CH, dw), F32),
            pltpu.SemaphoreType.DMA,
            pltpu.SemaphoreType.DMA,
        ],
    )
    def k(gs_hbm, gr_hbm, send_hbm, rec_hbm, h_hbm,
          sidx, ridx, bufs, bufr, sem_s, sem_r):
        wid = lax.axis_index("s") * NC + lax.axis_index("c")
        base = wid * epw

        def chunk(i, carry):
            off = base + i * CH
            pltpu.sync_copy(send_hbm.at[pl.ds(off, CH)], sidx)
            pltpu.sync_copy(rec_hbm.at[pl.ds(off, CH)], ridx)
            cs = pltpu.async_copy(gs_hbm.at[sidx], bufs, sem_s)
            cr = pltpu.async_copy(gr_hbm.at[ridx], bufr, sem_r)
            cs.wait()
            cr.wait()

            def addrow(r, c2):
                for j in range(dw // 16):
                    sl = pl.ds(j * 16, 16)
                    bufs[r, sl] = bufs[r, sl] + bufr[r, sl]
                return c2

            lax.fori_loop(0, CH, addrow, 0)
            pltpu.sync_copy(bufs, h_hbm.at[pl.ds(off, CH)])
            return carry

        lax.fori_loop(0, nchunk, chunk, 0)

    return k


def _edge_body(h_ref, w1c_ref, w2_ref, b2_ref, m_ref):
    hb = h_ref[...]
    d = w2_ref.shape[0]
    hpart = hb[:, :d]
    dz = hb[:, d:]
    dist = jnp.sqrt(jnp.sum(dz * dz, axis=1, keepdims=True) + 1e-12)
    h = jax.nn.silu(hpart + dist * w1c_ref[...])
    t = jnp.dot(h, w2_ref[...], preferred_element_type=F32) + b2_ref[...]
    m_ref[...] = jax.nn.silu(t)


def _edge_mlp(h, w1c, w2T, b2):
    e_pad, dw = h.shape
    d = w2T.shape[0]
    be = 512
    grid = e_pad // be
    return pl.pallas_call(
        _edge_body,
        grid=(grid,),
        in_specs=[
            pl.BlockSpec((be, dw), lambda i: (i, 0)),
            pl.BlockSpec((1, d), lambda i: (0, 0)),
            pl.BlockSpec((d, d), lambda i: (0, 0)),
            pl.BlockSpec((1, d), lambda i: (0, 0)),
        ],
        out_specs=pl.BlockSpec((be, d), lambda i: (i, 0)),
        out_shape=jax.ShapeDtypeStruct((e_pad, d), F32),
    )(h, w1c, w2T, b2)


def _agg_kernel(e_pad, n_sh, d):
    epw = e_pad // NW
    nchunk = epw // CH
    rows_per_tile = n_sh // NS
    ozchunk = rows_per_tile // CH
    mesh = plsc.VectorSubcoreMesh(
        core_axis_name="c", subcore_axis_name="s",
        num_cores=NC, num_subcores=NS)

    @functools.partial(
        pl.kernel,
        out_type=jax.ShapeDtypeStruct((NC, n_sh, d), F32),
        mesh=mesh,
        scratch_types=[
            pltpu.VMEM((CH,), jnp.int32),
            pltpu.VMEM((CH, d), F32),
            pltpu.VMEM_SHARED((n_sh, d), F32),
        ],
    )
    def k(rec_hbm, m_hbm, out_hbm, ridx, mbuf, shared):
        cid = lax.axis_index("c")
        sid = lax.axis_index("s")
        wid = sid * NC + cid
        tbase = sid * rows_per_tile

        def zrow(r, c2):
            for j in range(d // 16):
                mbuf[r, pl.ds(j * 16, 16)] = jnp.zeros((16,), F32)
            return c2

        lax.fori_loop(0, CH, zrow, 0)

        def zchunk(i, c2):
            pltpu.sync_copy(mbuf, shared.at[pl.ds(tbase + i * CH, CH)])
            return c2

        lax.fori_loop(0, ozchunk, zchunk, 0)
        plsc.subcore_barrier()

        base = wid * epw

        def chunk(i, c2):
            off = base + i * CH
            pltpu.sync_copy(rec_hbm.at[pl.ds(off, CH)], ridx)
            pltpu.sync_copy(m_hbm.at[pl.ds(off, CH)], mbuf)
            pltpu.sync_copy(mbuf, shared.at[ridx], add=True)
            return c2

        lax.fori_loop(0, nchunk, chunk, 0)
        plsc.subcore_barrier()

        def ochunk(i, c2):
            sl = pl.ds(tbase + i * CH, CH)
            pltpu.sync_copy(shared.at[sl], out_hbm.at[cid, sl])
            return c2

        lax.fori_loop(0, ozchunk, ochunk, 0)

    return k


def _node_body(x_ref, p0_ref, p1_ref, w3a_ref, w3b_ref, b3_ref,
               w4_ref, b4_ref, out_ref):
    xb = x_ref[...]
    aggr = p0_ref[...] + p1_ref[...]
    u = jax.nn.silu(
        jnp.dot(xb, w3a_ref[...], preferred_element_type=F32)
        + jnp.dot(aggr, w3b_ref[...], preferred_element_type=F32)
        + b3_ref[...])
    out_ref[...] = jnp.dot(u, w4_ref[...], preferred_element_type=F32) \
        + b4_ref[...]


def _node_mlp(x, p0, p1, w3aT, w3bT, b3, w4T, b4):
    n, d = x.shape
    bn = 1000
    grid = n // bn
    return pl.pallas_call(
        _node_body,
        grid=(grid,),
        in_specs=[
            pl.BlockSpec((bn, d), lambda i: (i, 0)),
            pl.BlockSpec((bn, d), lambda i: (i, 0)),
            pl.BlockSpec((bn, d), lambda i: (i, 0)),
            pl.BlockSpec((d, d), lambda i: (0, 0)),
            pl.BlockSpec((d, d), lambda i: (0, 0)),
            pl.BlockSpec((1, d), lambda i: (0, 0)),
            pl.BlockSpec((d, d), lambda i: (0, 0)),
            pl.BlockSpec((1, d), lambda i: (0, 0)),
        ],
        out_specs=pl.BlockSpec((bn, d), lambda i: (i, 0)),
        out_shape=jax.ShapeDtypeStruct((n, d), F32),
    )(x, p0, p1, w3aT, w3bT, b3, w4T, b4)


def kernel(x, pos, edge_index, W1, b1, W2, b2, W3, b3, W4, b4):
    n, d = x.shape
    e = edge_index.shape[1]
    dw = d + 16
    e_pad = _round_up(e, NW * CH)
    n_sh = _round_up(n + 1, NS * CH)

    waT = W1[:, :d].T
    wbT = W1[:, d:2 * d].T
    w1c = W1[:, 2 * d].reshape(1, d)
    b1r = b1.reshape(1, d)
    w2T = W2.T
    b2r = b2.reshape(1, d)
    w3aT = W3[:, :d].T
    w3bT = W3[:, d:].T
    b3r = b3.reshape(1, d)
    w4T = W4.T
    b4r = b4.reshape(1, d)

    send = edge_index[0]
    rec = edge_index[1]
    pad = e_pad - e
    send_p = jnp.concatenate([send, jnp.zeros((pad,), jnp.int32)])
    rec_p = jnp.concatenate([rec, jnp.zeros((pad,), jnp.int32)])
    rec_agg = jnp.concatenate([rec, jnp.full((pad,), n, jnp.int32)])

    gs, gr = _build_tables(x, pos, waT, wbT, b1r)
    h = _gather_kernel(e_pad, n, dw)(gs, gr, send_p, rec_p)
    msgs = _edge_mlp(h, w1c, w2T, b2r)
    partials = _agg_kernel(e_pad, n_sh, d)(rec_agg, msgs)
    update = _node_mlp(x, partials[0, :n], partials[1, :n],
                       w3aT, w3bT, b3r, w4T, b4r)
    return update


# speedup vs baseline: 2.6571x; 2.6571x over previous
import functools

import jax
import jax.numpy as jnp
from jax import lax
from jax.experimental import pallas as pl
from jax.experimental.pallas import tpu as pltpu
from jax.experimental.pallas import tpu_sc as plsc

F32 = jnp.float32

NC = 2
NS = 16
NW = NC * NS
CH = 128


def _round_up(a, m):
    return (a + m - 1) // m * m


def _tables_body(x_ref, pos_ref, wa_ref, wb_ref, b1_ref, gs_ref, gr_ref):
    xb = x_ref[...]
    bn = xb.shape[0]
    pz = jnp.concatenate(
        [pos_ref[...], jnp.zeros((bn, 125), F32)], axis=1)
    ms = jnp.dot(xb, wa_ref[...], preferred_element_type=F32)
    mr = jnp.dot(xb, wb_ref[...], preferred_element_type=F32) + b1_ref[...]
    gs_ref[...] = jnp.concatenate([ms, pz], axis=1)
    gr_ref[...] = jnp.concatenate([mr, -pz], axis=1)


def _build_tables(x, pos, waT, wbT, b1):
    n, d = x.shape
    bn = 1000
    grid = n // bn
    return pl.pallas_call(
        _tables_body,
        grid=(grid,),
        in_specs=[
            pl.BlockSpec((bn, d), lambda i: (i, 0)),
            pl.BlockSpec((bn, 3), lambda i: (i, 0)),
            pl.BlockSpec((d, d), lambda i: (0, 0)),
            pl.BlockSpec((d, d), lambda i: (0, 0)),
            pl.BlockSpec((1, d), lambda i: (0, 0)),
        ],
        out_specs=[
            pl.BlockSpec((bn, 2 * d), lambda i: (i, 0)),
            pl.BlockSpec((bn, 2 * d), lambda i: (i, 0)),
        ],
        out_shape=[
            jax.ShapeDtypeStruct((n, 2 * d), F32),
            jax.ShapeDtypeStruct((n, 2 * d), F32),
        ],
    )(x, pos, waT, wbT, b1)


def _gather_kernel(e_pad, n, dw):
    epw = e_pad // NW
    nchunk = epw // CH
    mesh = plsc.VectorSubcoreMesh(
        core_axis_name="c", subcore_axis_name="s",
        num_cores=NC, num_subcores=NS)

    @functools.partial(
        pl.kernel,
        out_type=jax.ShapeDtypeStruct((e_pad, dw), F32),
        mesh=mesh,
        scratch_types=[
            pltpu.VMEM((CH,), jnp.int32),
            pltpu.VMEM((CH,), jnp.int32),
            pltpu.VMEM((CH, dw), F32),
            pltpu.VMEM((CH, dw), F32),
            pltpu.SemaphoreType.DMA,
            pltpu.SemaphoreType.DMA,
        ],
    )
    def k(gs_hbm, gr_hbm, send_hbm, rec_hbm, h_hbm,
          sidx, ridx, bufs, bufr, sem_s, sem_r):
        wid = lax.axis_index("s") * NC + lax.axis_index("c")
        base = wid * epw

        def chunk(i, carry):
            off = base + i * CH
            pltpu.sync_copy(send_hbm.at[pl.ds(off, CH)], sidx)
            pltpu.sync_copy(rec_hbm.at[pl.ds(off, CH)], ridx)
            cs = pltpu.async_copy(gs_hbm.at[sidx], bufs, sem_s)
            cr = pltpu.async_copy(gr_hbm.at[ridx], bufr, sem_r)
            cs.wait()
            cr.wait()

            def addrow(r, c2):
                for j in range(9):
                    sl = pl.ds(j * 16, 16)
                    bufs[r, sl] = bufs[r, sl] + bufr[r, sl]
                return c2

            lax.fori_loop(0, CH, addrow, 0)
            pltpu.sync_copy(bufs, h_hbm.at[pl.ds(off, CH)])
            return carry

        lax.fori_loop(0, nchunk, chunk, 0)

    return k


def _edge_body(h_ref, w1c_ref, w2_ref, b2_ref, m_ref):
    hb = h_ref[...]
    d = w2_ref.shape[0]
    hpart = hb[:, :d]
    dz = hb[:, d:]
    dist = jnp.sqrt(jnp.sum(dz * dz, axis=1, keepdims=True) + 1e-12)
    h = jax.nn.silu(hpart + dist * w1c_ref[...])
    t = jnp.dot(h, w2_ref[...], preferred_element_type=F32) + b2_ref[...]
    m_ref[...] = jax.nn.silu(t)


def _edge_mlp(h, w1c, w2T, b2):
    e_pad, dw = h.shape
    d = w2T.shape[0]
    be = 512
    grid = e_pad // be
    return pl.pallas_call(
        _edge_body,
        grid=(grid,),
        in_specs=[
            pl.BlockSpec((be, dw), lambda i: (i, 0)),
            pl.BlockSpec((1, d), lambda i: (0, 0)),
            pl.BlockSpec((d, d), lambda i: (0, 0)),
            pl.BlockSpec((1, d), lambda i: (0, 0)),
        ],
        out_specs=pl.BlockSpec((be, d), lambda i: (i, 0)),
        out_shape=jax.ShapeDtypeStruct((e_pad, d), F32),
    )(h, w1c, w2T, b2)


def _agg_kernel(e_pad, n_sh, d):
    epw = e_pad // NW
    nchunk = epw // CH
    rows_per_tile = n_sh // NS
    ozchunk = rows_per_tile // CH
    mesh = plsc.VectorSubcoreMesh(
        core_axis_name="c", subcore_axis_name="s",
        num_cores=NC, num_subcores=NS)

    @functools.partial(
        pl.kernel,
        out_type=jax.ShapeDtypeStruct((NC, n_sh, d), F32),
        mesh=mesh,
        scratch_types=[
            pltpu.VMEM((CH,), jnp.int32),
            pltpu.VMEM((CH, d), F32),
            pltpu.VMEM_SHARED((n_sh, d), F32),
        ],
    )
    def k(rec_hbm, m_hbm, out_hbm, ridx, mbuf, shared):
        cid = lax.axis_index("c")
        sid = lax.axis_index("s")
        wid = sid * NC + cid
        tbase = sid * rows_per_tile

        def zrow(r, c2):
            for j in range(d // 16):
                mbuf[r, pl.ds(j * 16, 16)] = jnp.zeros((16,), F32)
            return c2

        lax.fori_loop(0, CH, zrow, 0)

        def zchunk(i, c2):
            pltpu.sync_copy(mbuf, shared.at[pl.ds(tbase + i * CH, CH)])
            return c2

        lax.fori_loop(0, ozchunk, zchunk, 0)
        plsc.subcore_barrier()

        base = wid * epw

        def chunk(i, c2):
            off = base + i * CH
            pltpu.sync_copy(rec_hbm.at[pl.ds(off, CH)], ridx)
            pltpu.sync_copy(m_hbm.at[pl.ds(off, CH)], mbuf)
            pltpu.sync_copy(mbuf, shared.at[ridx], add=True)
            return c2

        lax.fori_loop(0, nchunk, chunk, 0)
        plsc.subcore_barrier()

        def ochunk(i, c2):
            sl = pl.ds(tbase + i * CH, CH)
            pltpu.sync_copy(shared.at[sl], out_hbm.at[cid, sl])
            return c2

        lax.fori_loop(0, ozchunk, ochunk, 0)

    return k


def _node_body(x_ref, p0_ref, p1_ref, w3a_ref, w3b_ref, b3_ref,
               w4_ref, b4_ref, out_ref):
    xb = x_ref[...]
    aggr = p0_ref[...] + p1_ref[...]
    u = jax.nn.silu(
        jnp.dot(xb, w3a_ref[...], preferred_element_type=F32)
        + jnp.dot(aggr, w3b_ref[...], preferred_element_type=F32)
        + b3_ref[...])
    out_ref[...] = jnp.dot(u, w4_ref[...], preferred_element_type=F32) \
        + b4_ref[...]


def _node_mlp(x, p0, p1, w3aT, w3bT, b3, w4T, b4):
    n, d = x.shape
    bn = 1000
    grid = n // bn
    return pl.pallas_call(
        _node_body,
        grid=(grid,),
        in_specs=[
            pl.BlockSpec((bn, d), lambda i: (i, 0)),
            pl.BlockSpec((bn, d), lambda i: (i, 0)),
            pl.BlockSpec((bn, d), lambda i: (i, 0)),
            pl.BlockSpec((d, d), lambda i: (0, 0)),
            pl.BlockSpec((d, d), lambda i: (0, 0)),
            pl.BlockSpec((1, d), lambda i: (0, 0)),
            pl.BlockSpec((d, d), lambda i: (0, 0)),
            pl.BlockSpec((1, d), lambda i: (0, 0)),
        ],
        out_specs=pl.BlockSpec((bn, d), lambda i: (i, 0)),
        out_shape=jax.ShapeDtypeStruct((n, d), F32),
    )(x, p0, p1, w3aT, w3bT, b3, w4T, b4)


def kernel(x, pos, edge_index, W1, b1, W2, b2, W3, b3, W4, b4):
    n, d = x.shape
    e = edge_index.shape[1]
    dw = 2 * d
    e_pad = _round_up(e, NW * CH)
    n_sh = _round_up(n + 1, NS * CH)

    waT = W1[:, :d].T
    wbT = W1[:, d:2 * d].T
    w1c = W1[:, 2 * d].reshape(1, d)
    b1r = b1.reshape(1, d)
    w2T = W2.T
    b2r = b2.reshape(1, d)
    w3aT = W3[:, :d].T
    w3bT = W3[:, d:].T
    b3r = b3.reshape(1, d)
    w4T = W4.T
    b4r = b4.reshape(1, d)

    send = edge_index[0]
    rec = edge_index[1]
    pad = e_pad - e
    send_p = jnp.concatenate([send, jnp.zeros((pad,), jnp.int32)])
    rec_p = jnp.concatenate([rec, jnp.zeros((pad,), jnp.int32)])
    rec_agg = jnp.concatenate([rec, jnp.full((pad,), n, jnp.int32)])

    gs, gr = _build_tables(x, pos, waT, wbT, b1r)
    h = _gather_kernel(e_pad, n, dw)(gs, gr, send_p, rec_p)
    msgs = _edge_mlp(h, w1c, w2T, b2r)
    partials = _agg_kernel(e_pad, n_sh, d)(rec_agg, msgs)
    update = _node_mlp(x, partials[0, :n], partials[1, :n],
                       w3aT, w3bT, b3r, w4T, b4r)
    return update


# speedup vs baseline: 2.8430x; 1.0700x over previous
import functools

import jax
import jax.numpy as jnp
from jax import lax
from jax.experimental import pallas as pl
from jax.experimental.pallas import tpu as pltpu
from jax.experimental.pallas import tpu_sc as plsc

F32 = jnp.float32

NC = 2
NS = 16
NW = NC * NS
CH = 128


def _round_up(a, m):
    return (a + m - 1) // m * m


def _tables_body(x_ref, pos_ref, wa_ref, wb_ref, b1_ref, gs_ref, gr_ref):
    xb = x_ref[...]
    bn = xb.shape[0]
    pz = jnp.concatenate(
        [pos_ref[...], jnp.zeros((bn, 125), F32)], axis=1)
    ms = jnp.dot(xb, wa_ref[...], preferred_element_type=F32)
    mr = jnp.dot(xb, wb_ref[...], preferred_element_type=F32) + b1_ref[...]
    gs_ref[...] = jnp.concatenate([ms, pz], axis=1)
    gr_ref[...] = jnp.concatenate([mr, -pz], axis=1)


def _build_tables(x, pos, waT, wbT, b1):
    n, d = x.shape
    bn = 1000
    grid = n // bn
    return pl.pallas_call(
        _tables_body,
        grid=(grid,),
        in_specs=[
            pl.BlockSpec((bn, d), lambda i: (i, 0)),
            pl.BlockSpec((bn, 3), lambda i: (i, 0)),
            pl.BlockSpec((d, d), lambda i: (0, 0)),
            pl.BlockSpec((d, d), lambda i: (0, 0)),
            pl.BlockSpec((1, d), lambda i: (0, 0)),
        ],
        out_specs=[
            pl.BlockSpec((bn, 2 * d), lambda i: (i, 0)),
            pl.BlockSpec((bn, 2 * d), lambda i: (i, 0)),
        ],
        out_shape=[
            jax.ShapeDtypeStruct((n, 2 * d), F32),
            jax.ShapeDtypeStruct((n, 2 * d), F32),
        ],
    )(x, pos, waT, wbT, b1)


def _gather_kernel(e_pad, n, dw):
    bch = 64
    epw = e_pad // NW
    nchunk = epw // bch
    npair = nchunk // 2
    mesh = plsc.VectorSubcoreMesh(
        core_axis_name="c", subcore_axis_name="s",
        num_cores=NC, num_subcores=NS)

    @functools.partial(
        pl.kernel,
        out_type=jax.ShapeDtypeStruct((e_pad, dw), F32),
        mesh=mesh,
        scratch_types=[
            pltpu.VMEM((bch,), jnp.int32),
            pltpu.VMEM((bch,), jnp.int32),
            pltpu.VMEM((bch, dw), F32),
            pltpu.VMEM((bch, dw), F32),
            pltpu.VMEM((bch,), jnp.int32),
            pltpu.VMEM((bch,), jnp.int32),
            pltpu.VMEM((bch, dw), F32),
            pltpu.VMEM((bch, dw), F32),
            pltpu.SemaphoreType.DMA,
            pltpu.SemaphoreType.DMA,
            pltpu.SemaphoreType.DMA,
            pltpu.SemaphoreType.DMA,
        ],
    )
    def k(gs_hbm, gr_hbm, send_hbm, rec_hbm, h_hbm,
          sidx0, ridx0, bs0, br0, sidx1, ridx1, bs1, br1, g0, g1, s0, s1):
        wid = lax.axis_index("s") * NC + lax.axis_index("c")
        base = wid * epw

        def addstore(bs, br, off, sem):
            def addrow(r, c2):
                for j in range(9):
                    sl = pl.ds(j * 16, 16)
                    bs[r, sl] = bs[r, sl] + br[r, sl]
                return c2

            lax.fori_loop(0, bch, addrow, 0)
            pltpu.async_copy(bs, h_hbm.at[pl.ds(off, bch)], sem)

        def pair(j, carry):
            off0 = base + (2 * j) * bch
            off1 = off0 + bch

            @pl.when(j > 0)
            def _drain():
                pltpu.make_async_copy(
                    bs0, h_hbm.at[pl.ds(base, bch)], s0).wait()
                pltpu.make_async_copy(
                    bs1, h_hbm.at[pl.ds(base, bch)], s1).wait()

            pltpu.sync_copy(send_hbm.at[pl.ds(off0, bch)], sidx0)
            pltpu.sync_copy(rec_hbm.at[pl.ds(off0, bch)], ridx0)
            pltpu.async_copy(gs_hbm.at[sidx0], bs0, g0)
            pltpu.async_copy(gr_hbm.at[ridx0], br0, g0)
            pltpu.sync_copy(send_hbm.at[pl.ds(off1, bch)], sidx1)
            pltpu.sync_copy(rec_hbm.at[pl.ds(off1, bch)], ridx1)
            pltpu.async_copy(gs_hbm.at[sidx1], bs1, g1)
            pltpu.async_copy(gr_hbm.at[ridx1], br1, g1)

            pltpu.make_async_copy(gs_hbm.at[sidx0], bs0, g0).wait()
            pltpu.make_async_copy(gr_hbm.at[ridx0], br0, g0).wait()
            addstore(bs0, br0, off0, s0)
            pltpu.make_async_copy(gs_hbm.at[sidx1], bs1, g1).wait()
            pltpu.make_async_copy(gr_hbm.at[ridx1], br1, g1).wait()
            addstore(bs1, br1, off1, s1)
            return carry

        lax.fori_loop(0, npair, pair, 0)
        pltpu.make_async_copy(bs0, h_hbm.at[pl.ds(base, bch)], s0).wait()
        pltpu.make_async_copy(bs1, h_hbm.at[pl.ds(base, bch)], s1).wait()

    return k


def _edge_body(h_ref, w1c_ref, w2_ref, b2_ref, m_ref):
    hb = h_ref[...]
    d = w2_ref.shape[0]
    hpart = hb[:, :d]
    dz = hb[:, d:]
    dist = jnp.sqrt(jnp.sum(dz * dz, axis=1, keepdims=True) + 1e-12)
    h = jax.nn.silu(hpart + dist * w1c_ref[...])
    t = jnp.dot(h, w2_ref[...], preferred_element_type=F32) + b2_ref[...]
    m_ref[...] = jax.nn.silu(t)


def _edge_mlp(h, w1c, w2T, b2):
    e_pad, dw = h.shape
    d = w2T.shape[0]
    be = 512
    grid = e_pad // be
    return pl.pallas_call(
        _edge_body,
        grid=(grid,),
        in_specs=[
            pl.BlockSpec((be, dw), lambda i: (i, 0)),
            pl.BlockSpec((1, d), lambda i: (0, 0)),
            pl.BlockSpec((d, d), lambda i: (0, 0)),
            pl.BlockSpec((1, d), lambda i: (0, 0)),
        ],
        out_specs=pl.BlockSpec((be, d), lambda i: (i, 0)),
        out_shape=jax.ShapeDtypeStruct((e_pad, d), F32),
    )(h, w1c, w2T, b2)


def _agg_kernel(e_pad, n_sh, d):
    epw = e_pad // NW
    nchunk = epw // CH
    rows_per_tile = n_sh // NS
    ozchunk = rows_per_tile // CH
    mesh = plsc.VectorSubcoreMesh(
        core_axis_name="c", subcore_axis_name="s",
        num_cores=NC, num_subcores=NS)

    @functools.partial(
        pl.kernel,
        out_type=jax.ShapeDtypeStruct((NC, n_sh, d), F32),
        mesh=mesh,
        scratch_types=[
            pltpu.VMEM((CH,), jnp.int32),
            pltpu.VMEM((CH, d), F32),
            pltpu.VMEM_SHARED((n_sh, d), F32),
        ],
    )
    def k(rec_hbm, m_hbm, out_hbm, ridx, mbuf, shared):
        cid = lax.axis_index("c")
        sid = lax.axis_index("s")
        wid = sid * NC + cid
        tbase = sid * rows_per_tile

        def zrow(r, c2):
            for j in range(d // 16):
                mbuf[r, pl.ds(j * 16, 16)] = jnp.zeros((16,), F32)
            return c2

        lax.fori_loop(0, CH, zrow, 0)

        def zchunk(i, c2):
            pltpu.sync_copy(mbuf, shared.at[pl.ds(tbase + i * CH, CH)])
            return c2

        lax.fori_loop(0, ozchunk, zchunk, 0)
        plsc.subcore_barrier()

        base = wid * epw

        def chunk(i, c2):
            off = base + i * CH
            pltpu.sync_copy(rec_hbm.at[pl.ds(off, CH)], ridx)
            pltpu.sync_copy(m_hbm.at[pl.ds(off, CH)], mbuf)
            pltpu.sync_copy(mbuf, shared.at[ridx], add=True)
            return c2

        lax.fori_loop(0, nchunk, chunk, 0)
        plsc.subcore_barrier()

        def ochunk(i, c2):
            sl = pl.ds(tbase + i * CH, CH)
            pltpu.sync_copy(shared.at[sl], out_hbm.at[cid, sl])
            return c2

        lax.fori_loop(0, ozchunk, ochunk, 0)

    return k


def _node_body(x_ref, p0_ref, p1_ref, w3a_ref, w3b_ref, b3_ref,
               w4_ref, b4_ref, out_ref):
    xb = x_ref[...]
    aggr = p0_ref[...] + p1_ref[...]
    u = jax.nn.silu(
        jnp.dot(xb, w3a_ref[...], preferred_element_type=F32)
        + jnp.dot(aggr, w3b_ref[...], preferred_element_type=F32)
        + b3_ref[...])
    out_ref[...] = jnp.dot(u, w4_ref[...], preferred_element_type=F32) \
        + b4_ref[...]


def _node_mlp(x, p0, p1, w3aT, w3bT, b3, w4T, b4):
    n, d = x.shape
    bn = 1000
    grid = n // bn
    return pl.pallas_call(
        _node_body,
        grid=(grid,),
        in_specs=[
            pl.BlockSpec((bn, d), lambda i: (i, 0)),
            pl.BlockSpec((bn, d), lambda i: (i, 0)),
            pl.BlockSpec((bn, d), lambda i: (i, 0)),
            pl.BlockSpec((d, d), lambda i: (0, 0)),
            pl.BlockSpec((d, d), lambda i: (0, 0)),
            pl.BlockSpec((1, d), lambda i: (0, 0)),
            pl.BlockSpec((d, d), lambda i: (0, 0)),
            pl.BlockSpec((1, d), lambda i: (0, 0)),
        ],
        out_specs=pl.BlockSpec((bn, d), lambda i: (i, 0)),
        out_shape=jax.ShapeDtypeStruct((n, d), F32),
    )(x, p0, p1, w3aT, w3bT, b3, w4T, b4)


def kernel(x, pos, edge_index, W1, b1, W2, b2, W3, b3, W4, b4):
    n, d = x.shape
    e = edge_index.shape[1]
    dw = 2 * d
    e_pad = _round_up(e, NW * CH)
    n_sh = _round_up(n + 1, NS * CH)

    waT = W1[:, :d].T
    wbT = W1[:, d:2 * d].T
    w1c = W1[:, 2 * d].reshape(1, d)
    b1r = b1.reshape(1, d)
    w2T = W2.T
    b2r = b2.reshape(1, d)
    w3aT = W3[:, :d].T
    w3bT = W3[:, d:].T
    b3r = b3.reshape(1, d)
    w4T = W4.T
    b4r = b4.reshape(1, d)

    send = edge_index[0]
    rec = edge_index[1]
    pad = e_pad - e
    send_p = jnp.concatenate([send, jnp.zeros((pad,), jnp.int32)])
    rec_p = jnp.concatenate([rec, jnp.zeros((pad,), jnp.int32)])
    rec_agg = jnp.concatenate([rec, jnp.full((pad,), n, jnp.int32)])

    gs, gr = _build_tables(x, pos, waT, wbT, b1r)
    h = _gather_kernel(e_pad, n, dw)(gs, gr, send_p, rec_p)
    msgs = _edge_mlp(h, w1c, w2T, b2r)
    partials = _agg_kernel(e_pad, n_sh, d)(rec_agg, msgs)
    update = _node_mlp(x, partials[0, :n], partials[1, :n],
                       w3aT, w3bT, b3r, w4T, b4r)
    return update
